# baseline (speedup 1.0000x reference)
import functools
import math

import jax
import jax.numpy as jnp
from jax import lax
from jax.experimental import pallas as pl
from jax.experimental.pallas import tpu as pltpu


def _gate_tile(x, wcat, *, n_exp, rank):
    er = n_exp * rank
    y = jnp.dot(x, wcat, preferred_element_type=jnp.float32)

    col = lax.broadcasted_iota(jnp.int32, y.shape, 1)
    expert = (col & (er - 1)) // rank
    inv_expert = (n_exp - 1) - expert
    in_gate = col >= er
    int_min = jnp.int32(-2**31)

    u = lax.bitcast_convert_type(y, jnp.int32)
    key = u ^ ((u >> 31) & jnp.int32(0x7FFFFFFF))
    key = (key & jnp.int32(~(n_exp - 1))) | inv_expert
    key = jnp.where(in_gate, key, int_min)

    k1 = jnp.max(key, axis=-1, keepdims=True)
    sel1 = expert == ((n_exp - 1) - (k1 & (n_exp - 1)))
    key2 = jnp.where(sel1, int_min, key)
    k2 = jnp.max(key2, axis=-1, keepdims=True)
    sel2 = expert == ((n_exp - 1) - (k2 & (n_exp - 1)))

    def unkey(k):
        return lax.bitcast_convert_type(k ^ ((k >> 31) & jnp.int32(0x7FFFFFFF)),
                                        jnp.float32)

    e2 = jnp.exp(unkey(k2) - unkey(k1))
    inv = 1.0 / (1.0 + e2)
    w = jnp.where(sel1, inv, 0.0) + jnp.where(sel2, e2 * inv, 0.0)
    return (y * w).astype(jnp.bfloat16)


def _moe_body(x_ref, wcat_ref, bpad_ref, out_ref, g_ref, *, n_exp, rank,
              n_split):
    j = pl.program_id(1)

    @pl.when(j == 0)
    def _():
        g_ref[...] = _gate_tile(x_ref[...], wcat_ref[...], n_exp=n_exp,
                                rank=rank)

    nj = bpad_ref.shape[1] // n_split
    out_ref[...] = jnp.dot(g_ref[...], bpad_ref[:, pl.ds(j * nj, nj)],
                           preferred_element_type=jnp.float32)


@functools.partial(jax.jit, static_argnames=("n_exp", "rank", "interpret"))
def _moe(x, wcat, bpad, n_exp, rank, interpret=False):
    t, d = x.shape
    out_d = bpad.shape[1]
    tb = 1024
    n_split = 2
    body = functools.partial(_moe_body, n_exp=n_exp, rank=rank,
                             n_split=n_split)
    return pl.pallas_call(
        body,
        grid=(t // tb, n_split),
        in_specs=[
            pl.BlockSpec((tb, d), lambda i, j: (i, 0)),
            pl.BlockSpec((d, 2 * n_exp * rank), lambda i, j: (0, 0)),
            pl.BlockSpec((2 * n_exp * rank, out_d), lambda i, j: (0, 0)),
        ],
        out_specs=pl.BlockSpec((tb, out_d // n_split), lambda i, j: (i, j)),
        out_shape=jax.ShapeDtypeStruct((t, out_d), jnp.float32),
        scratch_shapes=[pltpu.VMEM((tb, 2 * n_exp * rank), jnp.bfloat16)],
        compiler_params=pltpu.CompilerParams(
            dimension_semantics=("arbitrary", "arbitrary"),
        ),
        interpret=interpret,
    )(x, wcat, bpad)


def kernel(inputs, Wg, A, B, interpret=False):
    bsz, seq, d = inputs.shape
    n_exp, rank, _ = A.shape
    out_d = B.shape[1]
    er = n_exp * rank
    scaling = math.sqrt(3.0 * 1.0 * d / rank)
    x = inputs.reshape(bsz * seq, d)
    aflat = A.transpose(2, 0, 1).reshape(d, er)
    wg_rep = jnp.repeat(Wg.T, rank, axis=1)
    wcat = jnp.concatenate([aflat, wg_rep], axis=1)
    bflat = B.transpose(0, 2, 1).reshape(er, out_d) * scaling
    bpad = jnp.concatenate([bflat, jnp.zeros_like(bflat)], axis=0).astype(jnp.bfloat16)
    out = _moe(x, wcat, bpad, n_exp, rank, interpret=interpret)
    return out.reshape(bsz, seq, out_d)


# speedup vs baseline: 1.3616x; 1.3616x over previous
import functools
import math

import jax
import jax.numpy as jnp
from jax import lax
from jax.experimental import pallas as pl
from jax.experimental.pallas import tpu as pltpu


def _moe_body(x_ref, wcat_ref, bpad_ref, out_hbm, obuf, sem, *, n_exp, rank,
              tb, nsteps):
    er = n_exp * rank
    i = pl.program_id(0)
    slot = lax.rem(i, 2)

    @pl.when(i >= 2)
    def _():
        pltpu.make_async_copy(obuf.at[slot],
                              out_hbm.at[pl.ds((i - 2) * tb, tb), :],
                              sem.at[slot]).wait()

    x = x_ref[...]
    y = jnp.dot(x, wcat_ref[...], preferred_element_type=jnp.float32)

    col = lax.broadcasted_iota(jnp.int32, y.shape, 1)
    expert = (col & (er - 1)) // rank
    inv_expert = (n_exp - 1) - expert
    in_gate = col >= er
    int_min = jnp.int32(-2**31)

    u = lax.bitcast_convert_type(y, jnp.int32)
    key = u ^ ((u >> 31) & jnp.int32(0x7FFFFFFF))
    key = (key & jnp.int32(~(n_exp - 1))) | inv_expert
    key = jnp.where(in_gate, key, int_min)

    k1 = jnp.max(key, axis=-1, keepdims=True)
    sel1 = expert == ((n_exp - 1) - (k1 & (n_exp - 1)))
    key2 = jnp.where(sel1, int_min, key)
    k2 = jnp.max(key2, axis=-1, keepdims=True)
    sel2 = expert == ((n_exp - 1) - (k2 & (n_exp - 1)))

    def unkey(k):
        return lax.bitcast_convert_type(k ^ ((k >> 31) & jnp.int32(0x7FFFFFFF)),
                                        jnp.float32)

    e2 = jnp.exp(unkey(k2) - unkey(k1))
    inv = 1.0 / (1.0 + e2)
    w = jnp.where(sel1, inv, 0.0) + jnp.where(sel2, e2 * inv, 0.0)
    g = (y * w).astype(jnp.bfloat16)
    obuf[slot] = jnp.dot(g, bpad_ref[...], preferred_element_type=jnp.float32)
    pltpu.make_async_copy(obuf.at[slot],
                          out_hbm.at[pl.ds(i * tb, tb), :],
                          sem.at[slot]).start()

    @pl.when(i == nsteps - 1)
    def _():
        pltpu.make_async_copy(obuf.at[1 - slot],
                              out_hbm.at[pl.ds((i - 1) * tb, tb), :],
                              sem.at[1 - slot]).wait()
        pltpu.make_async_copy(obuf.at[slot],
                              out_hbm.at[pl.ds(i * tb, tb), :],
                              sem.at[slot]).wait()


@functools.partial(jax.jit, static_argnames=("n_exp", "rank", "interpret"))
def _moe(x, wcat, bpad, n_exp, rank, interpret=False):
    t, d = x.shape
    out_d = bpad.shape[1]
    tb = 1024
    nsteps = t // tb
    body = functools.partial(_moe_body, n_exp=n_exp, rank=rank, tb=tb,
                             nsteps=nsteps)
    return pl.pallas_call(
        body,
        grid=(nsteps,),
        in_specs=[
            pl.BlockSpec((tb, d), lambda i: (i, 0)),
            pl.BlockSpec((d, 2 * n_exp * rank), lambda i: (0, 0)),
            pl.BlockSpec((2 * n_exp * rank, out_d), lambda i: (0, 0)),
        ],
        out_specs=pl.BlockSpec(memory_space=pltpu.MemorySpace.HBM),
        out_shape=jax.ShapeDtypeStruct((t, out_d), jnp.float32),
        scratch_shapes=[
            pltpu.VMEM((2, tb, out_d), jnp.float32),
            pltpu.SemaphoreType.DMA((2,)),
        ],
        compiler_params=pltpu.CompilerParams(
            dimension_semantics=("arbitrary",),
        ),
        interpret=interpret,
    )(x, wcat, bpad)


def kernel(inputs, Wg, A, B, interpret=False):
    bsz, seq, d = inputs.shape
    n_exp, rank, _ = A.shape
    out_d = B.shape[1]
    er = n_exp * rank
    scaling = math.sqrt(3.0 * 1.0 * d / rank)
    x = inputs.reshape(bsz * seq, d)
    aflat = A.transpose(2, 0, 1).reshape(d, er)
    wg_rep = jnp.repeat(Wg.T, rank, axis=1)
    wcat = jnp.concatenate([aflat, wg_rep], axis=1)
    bflat = B.transpose(0, 2, 1).reshape(er, out_d) * scaling
    bpad = jnp.concatenate([bflat, jnp.zeros_like(bflat)], axis=0).astype(jnp.bfloat16)
    out = _moe(x, wcat, bpad, n_exp, rank, interpret=interpret)
    return out.reshape(bsz, seq, out_d)


# speedup vs baseline: 1.3874x; 1.0190x over previous
import functools
import math

import jax
import jax.numpy as jnp
from jax import lax
from jax.experimental import pallas as pl
from jax.experimental.pallas import tpu as pltpu


def _moe_body(x_ref, wcat_ref, bpad_ref, out_ref, *, n_exp, rank):
    er = n_exp * rank
    x = x_ref[...]
    y = jnp.dot(x, wcat_ref[...], preferred_element_type=jnp.float32)

    col = lax.broadcasted_iota(jnp.int32, y.shape, 1)
    expert = (col & (er - 1)) // rank
    inv_expert = (n_exp - 1) - expert
    in_gate = col >= er
    int_min = jnp.int32(-2**31)

    u = lax.bitcast_convert_type(y, jnp.int32)
    key = u ^ ((u >> 31) & jnp.int32(0x7FFFFFFF))
    key = (key & jnp.int32(~(n_exp - 1))) | inv_expert
    key = jnp.where(in_gate, key, int_min)

    k1 = jnp.max(key, axis=-1, keepdims=True)
    sel1 = expert == ((n_exp - 1) - (k1 & (n_exp - 1)))
    key2 = jnp.where(sel1, int_min, key)
    k2 = jnp.max(key2, axis=-1, keepdims=True)
    sel2 = expert == ((n_exp - 1) - (k2 & (n_exp - 1)))

    def unkey(k):
        return lax.bitcast_convert_type(k ^ ((k >> 31) & jnp.int32(0x7FFFFFFF)),
                                        jnp.float32)

    e2 = jnp.exp(unkey(k2) - unkey(k1))
    inv = 1.0 / (1.0 + e2)
    w = jnp.where(sel1, inv, 0.0) + jnp.where(sel2, e2 * inv, 0.0)
    g = (y * w).astype(jnp.bfloat16)
    out_ref[...] = jnp.dot(g, bpad_ref[...],
                           preferred_element_type=jnp.float32)


@functools.partial(jax.jit, static_argnames=("n_exp", "rank", "interpret"))
def _moe(x, wcat, bpad, n_exp, rank, interpret=False):
    t, d = x.shape
    out_d = bpad.shape[1]
    tb = 1024
    body = functools.partial(_moe_body, n_exp=n_exp, rank=rank)
    return pl.pallas_call(
        body,
        grid=(t // tb,),
        in_specs=[
            pl.BlockSpec((tb, d), lambda i: (i, 0)),
            pl.BlockSpec((d, 2 * n_exp * rank), lambda i: (0, 0)),
            pl.BlockSpec((2 * n_exp * rank, out_d), lambda i: (0, 0)),
        ],
        out_specs=pl.BlockSpec((tb, out_d), lambda i: (i, 0)),
        out_shape=jax.ShapeDtypeStruct((t, out_d), jnp.float32),
        compiler_params=pltpu.CompilerParams(
            dimension_semantics=("parallel",),
        ),
        interpret=interpret,
    )(x, wcat, bpad)


def kernel(inputs, Wg, A, B, interpret=False):
    bsz, seq, d = inputs.shape
    n_exp, rank, _ = A.shape
    out_d = B.shape[1]
    er = n_exp * rank
    scaling = math.sqrt(3.0 * 1.0 * d / rank)
    x = inputs.reshape(bsz * seq, d)
    aflat = A.transpose(2, 0, 1).reshape(d, er)
    wg_rep = jnp.repeat(Wg.T, rank, axis=1)
    wcat = jnp.concatenate([aflat, wg_rep], axis=1)
    bflat = B.transpose(0, 2, 1).reshape(er, out_d) * scaling
    bpad = jnp.concatenate([bflat, jnp.zeros_like(bflat)], axis=0).astype(jnp.bfloat16)
    out = _moe(x, wcat, bpad, n_exp, rank, interpret=interpret)
    return out.reshape(bsz, seq, out_d)
